# baseline (speedup 1.0000x reference)
import functools

import jax
import jax.numpy as jnp
from jax import lax
from jax.experimental import pallas as pl
from jax.experimental.pallas import tpu as pltpu
from jax.experimental.pallas import tpu_sc as plsc

B, C, N, NPOINT = 8, 128, 100000, 16384
NC, NS, L = 2, 16, 16
NW = NC * NS
WPB = NW // B
CPW = C // WPB
CHUNK = 4096
NCHUNK = NPOINT // CHUNK
VPC = CHUNK // L


def _gather_kernel(feat_hbm, idx_hbm, out_hbm, idx_v, row_v, obuf_v,
                   sem0, sem1, row_sem):
    wid = lax.axis_index("s") * NC + lax.axis_index("c")
    b = wid // WPB
    c0 = (wid % WPB) * CPW

    pltpu.sync_copy(idx_hbm.at[b], idx_v)

    sems = (sem0, sem1)
    pending = [None, None]
    NSPLIT = 4
    SEG = N // NSPLIT
    for ci in range(CPW):
        c = c0 + ci
        row_cps = []
        for s in range(NSPLIT):
            lo = s * SEG
            sz = SEG if s < NSPLIT - 1 else N - lo
            cp = pltpu.make_async_copy(
                feat_hbm.at[b, c, pl.ds(lo, sz)], row_v.at[pl.ds(lo, sz)],
                row_sem)
            cp.start()
            row_cps.append(cp)
        for cp in row_cps:
            cp.wait()
        for t in range(NCHUNK):
            sl = t % 2
            if pending[sl] is not None:
                pending[sl].wait()
                pending[sl] = None

            @plsc.parallel_loop(0, VPC, step=1, unroll=8)
            def body(jl, t=t, sl=sl):
                iv = idx_v[pl.ds(t * CHUNK + jl * L, L)]
                g = plsc.load_gather(row_v, [iv])
                obuf_v[sl, pl.ds(jl * L, L)] = g
            cp = pltpu.make_async_copy(
                obuf_v.at[sl], out_hbm.at[b, c, pl.ds(t * CHUNK, CHUNK)],
                sems[sl])
            cp.start()
            pending[sl] = cp
    for sl in range(2):
        if pending[sl] is not None:
            pending[sl].wait()


@jax.jit
def kernel(features, idx):
    mesh = plsc.VectorSubcoreMesh(core_axis_name="c", subcore_axis_name="s")
    run = functools.partial(
        pl.kernel,
        mesh=mesh,
        compiler_params=pltpu.CompilerParams(
            needs_layout_passes=False, use_tc_tiling_on_sc=False),
        out_type=jax.ShapeDtypeStruct((B, C, NPOINT), jnp.float32),
        scratch_types=[
            pltpu.VMEM((NPOINT,), jnp.int32),
            pltpu.VMEM((N,), jnp.float32),
            pltpu.VMEM((2, CHUNK), jnp.float32),
            pltpu.SemaphoreType.DMA,
            pltpu.SemaphoreType.DMA,
            pltpu.SemaphoreType.DMA,
        ],
    )(_gather_kernel)
    return run(features, idx)


# speedup vs baseline: 1.9199x; 1.9199x over previous
import functools

import jax
import jax.numpy as jnp
from jax import lax
from jax.experimental import pallas as pl
from jax.experimental.pallas import tpu as pltpu
from jax.experimental.pallas import tpu_sc as plsc

B, C, N, NPOINT = 8, 128, 100000, 16384
NC, NS, L = 2, 16, 16
NW = NC * NS
WPB = NW // B
CPW = C // WPB
CHUNK = 4096
NCHUNK = NPOINT // CHUNK
VPC = CHUNK // L


def _gather_kernel(feat_hbm, idx_hbm, out_hbm, idx_v, row_v, obuf_v,
                   sem0, sem1, row_sem):
    wid = lax.axis_index("s") * NC + lax.axis_index("c")
    b = wid // WPB
    c0 = (wid % WPB) * CPW

    pltpu.sync_copy(idx_hbm.at[b], idx_v)

    sems = (sem0, sem1)
    pending = [None, None]
    NSPLIT = 4
    SEG = N // NSPLIT
    for ci in range(CPW):
        c = c0 + ci
        cp = pltpu.make_async_copy(feat_hbm.at[b, c], row_v, row_sem)
        cp.start()
        cp.wait()
        for t in range(NCHUNK):
            sl = t % 2
            if pending[sl] is not None:
                pending[sl].wait()
                pending[sl] = None

            @plsc.parallel_loop(0, VPC, step=1, unroll=8)
            def body(jl, t=t, sl=sl):
                iv = idx_v[pl.ds(t * CHUNK + jl * L, L)]
                g = plsc.load_gather(row_v, [iv])
                obuf_v[sl, pl.ds(jl * L, L)] = g
            cp = pltpu.make_async_copy(
                obuf_v.at[sl], out_hbm.at[b, c, pl.ds(t * CHUNK, CHUNK)],
                sems[sl])
            cp.start()
            pending[sl] = cp
    for sl in range(2):
        if pending[sl] is not None:
            pending[sl].wait()


@jax.jit
def kernel(features, idx):
    mesh = plsc.VectorSubcoreMesh(core_axis_name="c", subcore_axis_name="s")
    run = functools.partial(
        pl.kernel,
        mesh=mesh,
        compiler_params=pltpu.CompilerParams(needs_layout_passes=False),
        out_type=jax.ShapeDtypeStruct((B, C, NPOINT), jnp.float32),
        scratch_types=[
            pltpu.VMEM((NPOINT,), jnp.int32),
            pltpu.VMEM((N,), jnp.float32),
            pltpu.VMEM((2, CHUNK), jnp.float32),
            pltpu.SemaphoreType.DMA,
            pltpu.SemaphoreType.DMA,
            pltpu.SemaphoreType.DMA,
        ],
    )(_gather_kernel)
    return run(features, idx)
